# baseline (speedup 1.0000x reference)
import jax
import jax.numpy as jnp
from jax import lax
from jax.experimental import pallas as pl
from jax.experimental.pallas import tpu as pltpu
from jax.experimental.pallas import tpu_sc as plsc

_NC = 2
_NS = 16
_NW = _NC * _NS
_L = 16
_CHUNK = 64
_NBUF = 2


def _gmf_body(user_hbm, item_hbm, ut3_hbm, it3_hbm, out_hbm,
              su_v, si_v, qu_v, qi_v, ut_tiles, it_tiles, out_stage,
              sem_u0, sem_u1, sem_i0, sem_i1, sem_o0, sem_o1):
    sems_u = (sem_u0, sem_u1)
    sems_i = (sem_i0, sem_i1)
    sems_o = (sem_o0, sem_o1)
    b_per_w = su_v.shape[0]
    n_chunks = b_per_w // _CHUNK
    wid = lax.axis_index("s") * _NC + lax.axis_index("c")
    base = wid * b_per_w

    pltpu.sync_copy(user_hbm.at[pl.ds(base, b_per_w)], su_v)
    pltpu.sync_copy(item_hbm.at[pl.ds(base, b_per_w)], si_v)

    def qcompute(j, _):
        u = su_v[pl.ds(j * _L, _L)]
        v = si_v[pl.ds(j * _L, _L)]
        qu_v[j // (_CHUNK // _L), pl.ds((j % (_CHUNK // _L)) * _L, _L)] = u >> 3
        qi_v[j // (_CHUNK // _L), pl.ds((j % (_CHUNK // _L)) * _L, _L)] = v >> 3
        return ()
    for j in range(b_per_w // _L):
        qcompute(j, ())

    def fire(c, buf):
        pltpu.async_copy(ut3_hbm.at[qu_v.at[c]], ut_tiles.at[buf],
                         sems_u[buf])
        pltpu.async_copy(it3_hbm.at[qi_v.at[c]], it_tiles.at[buf],
                         sems_i[buf])

    def drain(buf):
        pltpu.make_async_copy(ut3_hbm.at[qu_v.at[0]], ut_tiles.at[buf],
                              sems_u[buf]).wait()
        pltpu.make_async_copy(it3_hbm.at[qi_v.at[0]], it_tiles.at[buf],
                              sems_i[buf]).wait()

    lanes = lax.iota(jnp.int32, _L)

    def extract(c, buf):
        def per_grp(j16, _):
            slot = j16 * _L + lanes
            su = su_v[pl.ds(c * _CHUNK + j16 * _L, _L)] & 7
            si = si_v[pl.ds(c * _CHUNK + j16 * _L, _L)] & 7
            def per_f(f, _2):
                fv = jnp.full((_L,), f, jnp.int32)
                u = plsc.load_gather(ut_tiles.at[buf], [slot, su, fv])
                v = plsc.load_gather(it_tiles.at[buf], [slot, si, fv])
                plsc.store_scatter(out_stage.at[buf], [slot, fv], u * v)
                return ()
            lax.fori_loop(0, 32, per_f, ())
            return ()
        lax.fori_loop(0, _CHUNK // _L, per_grp, ())

    def flush(c, buf):
        pltpu.async_copy(out_stage.at[buf],
                         out_hbm.at[pl.ds(base + c * _CHUNK, _CHUNK)],
                         sems_o[buf])

    def drain_out(buf):
        pltpu.make_async_copy(out_stage.at[buf],
                              out_hbm.at[pl.ds(0, _CHUNK)],
                              sems_o[buf]).wait()

    fire(0, 0)

    def step(c2, _):
        for p in range(_NBUF):
            c = c2 * _NBUF + p
            nxt = c + 1
            @pl.when(nxt < n_chunks)
            def _():
                fire(nxt, (p + 1) % _NBUF)
            drain(p)
            @pl.when(c >= _NBUF)
            def _():
                drain_out(p)
            extract(c, p)
            flush(c, p)
        return ()
    lax.fori_loop(0, n_chunks // _NBUF, step, ())
    for p in range(_NBUF):
        drain_out(p)


@jax.jit
def kernel(user, item, user_table, item_table):
    b = user.shape[0]
    d = user_table.shape[1]
    b_per_w = b // _NW
    mesh = plsc.VectorSubcoreMesh(core_axis_name="c", subcore_axis_name="s")
    k = pl.kernel(
        _gmf_body,
        out_type=jax.ShapeDtypeStruct((b, d), jnp.float32),
        mesh=mesh,
        compiler_params=pltpu.CompilerParams(use_tc_tiling_on_sc=False,
                                             needs_layout_passes=False),
        scratch_types=[
            pltpu.VMEM((b_per_w,), jnp.int32),
            pltpu.VMEM((b_per_w,), jnp.int32),
            pltpu.VMEM((b_per_w // _CHUNK, _CHUNK), jnp.int32),
            pltpu.VMEM((b_per_w // _CHUNK, _CHUNK), jnp.int32),
            pltpu.VMEM((_NBUF, _CHUNK, 8, d), jnp.float32),
            pltpu.VMEM((_NBUF, _CHUNK, 8, d), jnp.float32),
            pltpu.VMEM((_NBUF, _CHUNK, d), jnp.float32),
            pltpu.SemaphoreType.DMA,
            pltpu.SemaphoreType.DMA,
            pltpu.SemaphoreType.DMA,
            pltpu.SemaphoreType.DMA,
            pltpu.SemaphoreType.DMA,
            pltpu.SemaphoreType.DMA,
        ],
    )
    return k(user.astype(jnp.int32), item.astype(jnp.int32),
             jnp.reshape(user_table, (user_table.shape[0] // 8, 8, d)),
             jnp.reshape(item_table, (item_table.shape[0] // 8, 8, d)))


# speedup vs baseline: 2.3353x; 2.3353x over previous
import jax
import jax.numpy as jnp
from jax import lax
from jax.experimental import pallas as pl
from jax.experimental.pallas import tpu as pltpu
from jax.experimental.pallas import tpu_sc as plsc

_NC = 2
_NS = 16
_NW = _NC * _NS
_L = 16
_CHUNK = 16
_NBUF = 2


def _gmf_body(user_hbm, item_hbm, ut_hbm, it_hbm, out_hbm,
              su_v, si_v, ut_tiles, it_tiles, out_stage,
              sem_u0, sem_u1, sem_i0, sem_i1, sem_o0, sem_o1):
    sems_u = (sem_u0, sem_u1)
    sems_i = (sem_i0, sem_i1)
    sems_o = (sem_o0, sem_o1)
    b_per_w = su_v.shape[0]
    n_chunks = b_per_w // _CHUNK
    wid = lax.axis_index("s") * _NC + lax.axis_index("c")
    base = wid * b_per_w

    ut3 = ut_hbm
    it3 = it_hbm
    out3 = out_hbm.reshape(out_hbm.shape[0] // 8, 8, 32)

    pltpu.sync_copy(user_hbm.at[pl.ds(base, b_per_w)], su_v)
    pltpu.sync_copy(item_hbm.at[pl.ds(base, b_per_w)], si_v)

    def fire(c, buf):
        qu_vec = su_v[pl.ds(c * _CHUNK, _L)] >> 3
        qi_vec = si_v[pl.ds(c * _CHUNK, _L)] >> 3
        for j in range(_CHUNK):
            pltpu.async_copy(ut3.at[pl.ds(qu_vec[j], 1)],
                             ut_tiles.at[buf].at[pl.ds(j, 1)], sems_u[buf])
            pltpu.async_copy(it3.at[pl.ds(qi_vec[j], 1)],
                             it_tiles.at[buf].at[pl.ds(j, 1)], sems_i[buf])

    def drain(buf):
        for j in range(_CHUNK):
            pltpu.make_async_copy(ut3.at[pl.ds(0, 1)],
                                  ut_tiles.at[buf].at[pl.ds(j, 1)],
                                  sems_u[buf]).wait()
            pltpu.make_async_copy(it3.at[pl.ds(0, 1)],
                                  it_tiles.at[buf].at[pl.ds(j, 1)],
                                  sems_i[buf]).wait()

    lanes = lax.iota(jnp.int32, _L)

    def extract(c, buf):
        su = su_v[pl.ds(c * _CHUNK, _L)] & 7
        si = si_v[pl.ds(c * _CHUNK, _L)] & 7
        g = lanes >> 3
        s = lanes & 7
        def per_f(f, _):
            fv = jnp.full((_L,), f, jnp.int32)
            u = plsc.load_gather(ut_tiles.at[buf], [lanes, su, fv])
            v = plsc.load_gather(it_tiles.at[buf], [lanes, si, fv])
            plsc.store_scatter(out_stage.at[buf], [g, s, fv], u * v)
            return ()
        lax.fori_loop(0, 32, per_f, ())

    def flush(c, buf):
        pltpu.async_copy(out_stage.at[buf],
                         out3.at[pl.ds(base // 8 + c * (_CHUNK // 8),
                                       _CHUNK // 8)], sems_o[buf])

    def drain_out(buf):
        pltpu.make_async_copy(out_stage.at[buf],
                              out3.at[pl.ds(0, _CHUNK // 8)],
                              sems_o[buf]).wait()

    fire(0, 0)

    def step(c2, _):
        for p in range(_NBUF):
            c = c2 * _NBUF + p
            nxt = c + 1
            @pl.when(nxt < n_chunks)
            def _():
                fire(nxt, (p + 1) % _NBUF)
            drain(p)
            @pl.when(c >= _NBUF)
            def _():
                drain_out(p)
            extract(c, p)
            flush(c, p)
        return ()
    lax.fori_loop(0, n_chunks // _NBUF, step, ())
    for p in range(_NBUF):
        drain_out(p)


@jax.jit
def kernel(user, item, user_table, item_table):
    b = user.shape[0]
    d = user_table.shape[1]
    b_per_w = b // _NW
    mesh = plsc.VectorSubcoreMesh(core_axis_name="c", subcore_axis_name="s")
    k = pl.kernel(
        _gmf_body,
        out_type=jax.ShapeDtypeStruct((b, d), jnp.float32),
        mesh=mesh,
        compiler_params=pltpu.CompilerParams(use_tc_tiling_on_sc=True,
                                             needs_layout_passes=False),
        scratch_types=[
            pltpu.VMEM((b_per_w,), jnp.int32),
            pltpu.VMEM((b_per_w,), jnp.int32),
            pltpu.VMEM((_NBUF, _CHUNK, 8, 32), jnp.float32),
            pltpu.VMEM((_NBUF, _CHUNK, 8, 32), jnp.float32),
            pltpu.VMEM((_NBUF, _CHUNK // 8, 8, 32), jnp.float32),
            pltpu.SemaphoreType.DMA,
            pltpu.SemaphoreType.DMA,
            pltpu.SemaphoreType.DMA,
            pltpu.SemaphoreType.DMA,
            pltpu.SemaphoreType.DMA,
            pltpu.SemaphoreType.DMA,
        ],
    )
    return k(user.astype(jnp.int32), item.astype(jnp.int32),
             jnp.reshape(user_table, (user_table.shape[0] // 8, 8, d)),
             jnp.reshape(item_table, (item_table.shape[0] // 8, 8, d)))
